# baseline (speedup 1.0000x reference)
import functools

import jax
import jax.numpy as jnp
from jax import lax
from jax.experimental import pallas as pl
from jax.experimental.pallas import tpu as pltpu
from jax.experimental.pallas import tpu_sc as plsc

_LANES = 16
_NWORKERS = 32
_CHUNK = 32768
_SPLIT = 26624
_NBUF = 2
_UNROLL = 16
_NACC = 8


def _sc_partial_sums(atomic_numbers, table_bcast, table_pad):
    natoms = atomic_numbers.shape[0]
    nrefs = table_bcast.shape[0] // _LANES
    npad = table_pad.shape[0]
    per_w = natoms // _NWORKERS
    nchunks = per_w // _CHUNK
    nstream = _CHUNK - _SPLIT
    assert per_w % _CHUNK == 0 and nchunks >= 2

    mesh = plsc.VectorSubcoreMesh(core_axis_name="c", subcore_axis_name="s")

    @functools.partial(
        pl.kernel,
        mesh=mesh,
        out_type=jax.ShapeDtypeStruct((_NWORKERS, _LANES), jnp.float32),
        compiler_params=pltpu.CompilerParams(
            use_tc_tiling_on_sc=False, needs_layout_passes=False
        ),
        scratch_types=[
            [pltpu.VMEM((_CHUNK,), jnp.int32) for _ in range(_NBUF)],
            [pltpu.VMEM((nstream,), jnp.float32) for _ in range(_NBUF)],
            pltpu.VMEM((nrefs * _LANES,), jnp.float32),
            pltpu.VMEM((nrefs * _LANES,), jnp.float32),
            pltpu.VMEM_SHARED((npad,), jnp.float32),
            pltpu.VMEM((_LANES,), jnp.float32),
            [pltpu.SemaphoreType.DMA for _ in range(_NBUF)],
            [pltpu.SemaphoreType.DMA for _ in range(_NBUF)],
        ],
    )
    def k(an_hbm, tabb_hbm, tabp_hbm, out_hbm, bufs, vals, hist, tab_v,
          tabs, acc_v, sems_i, sems_g):
        wid = lax.axis_index("s") * 2 + lax.axis_index("c")
        base = wid * per_w

        @pl.when(lax.axis_index("s") == 0)
        def _():
            pltpu.sync_copy(tabp_hbm, tabs)

        pltpu.sync_copy(tabb_hbm, tab_v)
        zeros = jnp.zeros((_LANES,), jnp.float32)
        for b in range(nrefs):
            hist[pl.ds(b * _LANES, _LANES)] = zeros
        plsc.subcore_barrier()

        lanes = lax.iota(jnp.int32, _LANES)
        ones = jnp.ones((_LANES,), jnp.float32)

        def start_idx(c):
            return pltpu.async_copy(
                an_hbm.at[pl.ds(base + c * _CHUNK, _CHUNK)],
                bufs[c % _NBUF],
                sems_i[c % _NBUF],
            )

        def start_gather(c):
            return pltpu.async_copy(
                tabs.at[bufs[c % _NBUF].at[pl.ds(_SPLIT, nstream)]],
                vals[c % _NBUF],
                sems_g[c % _NBUF],
            )

        def process_hist(buf):
            def body(i, carry):
                start_i = i * (_UNROLL * _LANES)
                idxs = [
                    buf[pl.ds(start_i + u * _LANES, _LANES)]
                    for u in range(_UNROLL)
                ]
                addrs = [ix * _LANES + lanes for ix in idxs]
                for a in addrs:
                    plsc.addupdate_scatter(hist, [a], ones)
                return carry

            lax.fori_loop(0, _SPLIT // (_UNROLL * _LANES), body, 0)

        def accum(v, accs):
            def body(i, accs):
                loads = [
                    v[pl.ds((i * _NACC + u) * _LANES, _LANES)]
                    for u in range(_NACC)
                ]
                return tuple(a + x for a, x in zip(accs, loads))

            return lax.fori_loop(0, nstream // (_NACC * _LANES), body, accs)

        accs = tuple(jnp.zeros((_LANES,), jnp.float32) for _ in range(_NACC))

        hidx = [start_idx(0)]
        hg = {}
        for c in range(nchunks):
            if c + 1 < nchunks:
                hidx.append(start_idx(c + 1))
            hidx[c].wait()
            hg[c] = start_gather(c)
            process_hist(bufs[c % _NBUF])
            hg[c].wait()
            accs = accum(vals[c % _NBUF], accs)

        acc = accs[0]
        for a in accs[1:]:
            acc = acc + a
        for b in range(nrefs):
            acc = acc + hist[pl.ds(b * _LANES, _LANES)] * tab_v[pl.ds(b * _LANES, _LANES)]
        acc_v[...] = acc
        pltpu.sync_copy(acc_v, out_hbm.at[wid])

    return k(atomic_numbers, table_bcast, table_pad)


def _tc_combine(partials_ref, te_ref, out_ref):
    out_ref[...] = te_ref[...] - jnp.sum(partials_ref[...])


def kernel(total_energy, atomic_numbers, per_atom_references):
    an = atomic_numbers.astype(jnp.int32)
    nrefs = per_atom_references.shape[0]
    table_f32 = per_atom_references.astype(jnp.float32)
    table_bcast = jnp.broadcast_to(
        table_f32[:, None], (nrefs, _LANES)
    ).reshape(nrefs * _LANES)
    npad = -(-nrefs // _LANES) * _LANES
    table_pad = jnp.pad(table_f32, (0, npad - nrefs))

    partials = _sc_partial_sums(an, table_bcast, table_pad)

    return pl.pallas_call(
        _tc_combine,
        out_shape=jax.ShapeDtypeStruct(total_energy.shape, jnp.float32),
    )(partials, total_energy)


# speedup vs baseline: 1.0327x; 1.0327x over previous
import functools

import jax
import jax.numpy as jnp
from jax import lax
from jax.experimental import pallas as pl
from jax.experimental.pallas import tpu as pltpu
from jax.experimental.pallas import tpu_sc as plsc

_LANES = 16
_NWORKERS = 32
_CHUNK = 32768
_SPLIT = 30720
_NBUF = 2
_UNROLL = 16
_NACC = 8


def _sc_partial_sums(atomic_numbers, table_bcast, table_pad):
    natoms = atomic_numbers.shape[0]
    nrefs = table_bcast.shape[0] // _LANES
    npad = table_pad.shape[0]
    per_w = natoms // _NWORKERS
    nchunks = per_w // _CHUNK
    nstream = _CHUNK - _SPLIT
    assert per_w % _CHUNK == 0 and nchunks >= 2

    mesh = plsc.VectorSubcoreMesh(core_axis_name="c", subcore_axis_name="s")

    @functools.partial(
        pl.kernel,
        mesh=mesh,
        out_type=jax.ShapeDtypeStruct((_NWORKERS, _LANES), jnp.float32),
        compiler_params=pltpu.CompilerParams(
            use_tc_tiling_on_sc=False, needs_layout_passes=False
        ),
        scratch_types=[
            [pltpu.VMEM((_CHUNK,), jnp.int32) for _ in range(_NBUF)],
            [pltpu.VMEM((nstream,), jnp.float32) for _ in range(_NBUF)],
            pltpu.VMEM((nrefs * _LANES,), jnp.float32),
            pltpu.VMEM((nrefs * _LANES,), jnp.float32),
            pltpu.VMEM_SHARED((npad,), jnp.float32),
            pltpu.VMEM((_LANES,), jnp.float32),
            [pltpu.SemaphoreType.DMA for _ in range(_NBUF)],
            [pltpu.SemaphoreType.DMA for _ in range(_NBUF)],
        ],
    )
    def k(an_hbm, tabb_hbm, tabp_hbm, out_hbm, bufs, vals, hist, tab_v,
          tabs, acc_v, sems_i, sems_g):
        wid = lax.axis_index("s") * 2 + lax.axis_index("c")
        base = wid * per_w

        @pl.when(lax.axis_index("s") == 0)
        def _():
            pltpu.sync_copy(tabp_hbm, tabs)

        pltpu.sync_copy(tabb_hbm, tab_v)
        zeros = jnp.zeros((_LANES,), jnp.float32)
        for b in range(nrefs):
            hist[pl.ds(b * _LANES, _LANES)] = zeros
        plsc.subcore_barrier()

        lanes = lax.iota(jnp.int32, _LANES)
        ones = jnp.ones((_LANES,), jnp.float32)

        def start_idx(c):
            return pltpu.async_copy(
                an_hbm.at[pl.ds(base + c * _CHUNK, _CHUNK)],
                bufs[c % _NBUF],
                sems_i[c % _NBUF],
            )

        def start_gather(c):
            return pltpu.async_copy(
                tabs.at[bufs[c % _NBUF].at[pl.ds(_SPLIT, nstream)]],
                vals[c % _NBUF],
                sems_g[c % _NBUF],
            )

        def process_hist(buf):
            def body(i, carry):
                start_i = i * (_UNROLL * _LANES)
                idxs = [
                    buf[pl.ds(start_i + u * _LANES, _LANES)]
                    for u in range(_UNROLL)
                ]
                addrs = [ix * _LANES + lanes for ix in idxs]
                for a in addrs:
                    plsc.addupdate_scatter(hist, [a], ones)
                return carry

            lax.fori_loop(0, _SPLIT // (_UNROLL * _LANES), body, 0)

        def accum(v, accs):
            def body(i, accs):
                loads = [
                    v[pl.ds((i * _NACC + u) * _LANES, _LANES)]
                    for u in range(_NACC)
                ]
                return tuple(a + x for a, x in zip(accs, loads))

            return lax.fori_loop(0, nstream // (_NACC * _LANES), body, accs)

        accs = tuple(jnp.zeros((_LANES,), jnp.float32) for _ in range(_NACC))

        hidx = [start_idx(0)]
        hg = {}
        for c in range(nchunks):
            if c + 1 < nchunks:
                hidx.append(start_idx(c + 1))
            hidx[c].wait()
            hg[c] = start_gather(c)
            process_hist(bufs[c % _NBUF])
            hg[c].wait()
            accs = accum(vals[c % _NBUF], accs)

        acc = accs[0]
        for a in accs[1:]:
            acc = acc + a
        for b in range(nrefs):
            acc = acc + hist[pl.ds(b * _LANES, _LANES)] * tab_v[pl.ds(b * _LANES, _LANES)]
        acc_v[...] = acc
        pltpu.sync_copy(acc_v, out_hbm.at[wid])

    return k(atomic_numbers, table_bcast, table_pad)


def _tc_combine(partials_ref, te_ref, out_ref):
    out_ref[...] = te_ref[...] - jnp.sum(partials_ref[...])


def kernel(total_energy, atomic_numbers, per_atom_references):
    an = atomic_numbers.astype(jnp.int32)
    nrefs = per_atom_references.shape[0]
    table_f32 = per_atom_references.astype(jnp.float32)
    table_bcast = jnp.broadcast_to(
        table_f32[:, None], (nrefs, _LANES)
    ).reshape(nrefs * _LANES)
    npad = -(-nrefs // _LANES) * _LANES
    table_pad = jnp.pad(table_f32, (0, npad - nrefs))

    partials = _sc_partial_sums(an, table_bcast, table_pad)

    return pl.pallas_call(
        _tc_combine,
        out_shape=jax.ShapeDtypeStruct(total_energy.shape, jnp.float32),
    )(partials, total_energy)


# speedup vs baseline: 1.0348x; 1.0020x over previous
import functools

import jax
import jax.numpy as jnp
from jax import lax
from jax.experimental import pallas as pl
from jax.experimental.pallas import tpu as pltpu
from jax.experimental.pallas import tpu_sc as plsc

_LANES = 16
_NWORKERS = 32
_CHUNK = 32768
_SPLIT = 28672
_NBUF = 2
_UNROLL = 16
_NACC = 8


def _sc_partial_sums(atomic_numbers, table_bcast, table_pad):
    natoms = atomic_numbers.shape[0]
    nrefs = table_bcast.shape[0] // _LANES
    npad = table_pad.shape[0]
    per_w = natoms // _NWORKERS
    nchunks = per_w // _CHUNK
    nstream = _CHUNK - _SPLIT
    assert per_w % _CHUNK == 0 and nchunks >= 2

    mesh = plsc.VectorSubcoreMesh(core_axis_name="c", subcore_axis_name="s")

    @functools.partial(
        pl.kernel,
        mesh=mesh,
        out_type=jax.ShapeDtypeStruct((_NWORKERS, _LANES), jnp.float32),
        compiler_params=pltpu.CompilerParams(
            use_tc_tiling_on_sc=False, needs_layout_passes=False
        ),
        scratch_types=[
            [pltpu.VMEM((_CHUNK,), jnp.int32) for _ in range(_NBUF)],
            [pltpu.VMEM((nstream,), jnp.float32) for _ in range(_NBUF)],
            pltpu.VMEM((nrefs * _LANES,), jnp.float32),
            pltpu.VMEM((nrefs * _LANES,), jnp.float32),
            pltpu.VMEM_SHARED((npad,), jnp.float32),
            pltpu.VMEM((_LANES,), jnp.float32),
            [pltpu.SemaphoreType.DMA for _ in range(_NBUF)],
            [pltpu.SemaphoreType.DMA for _ in range(_NBUF)],
        ],
    )
    def k(an_hbm, tabb_hbm, tabp_hbm, out_hbm, bufs, vals, hist, tab_v,
          tabs, acc_v, sems_i, sems_g):
        wid = lax.axis_index("s") * 2 + lax.axis_index("c")
        base = wid * per_w

        @pl.when(lax.axis_index("s") == 0)
        def _():
            pltpu.sync_copy(tabp_hbm, tabs)

        pltpu.sync_copy(tabb_hbm, tab_v)
        zeros = jnp.zeros((_LANES,), jnp.float32)
        for b in range(nrefs):
            hist[pl.ds(b * _LANES, _LANES)] = zeros
        plsc.subcore_barrier()

        lanes = lax.iota(jnp.int32, _LANES)
        ones = jnp.ones((_LANES,), jnp.float32)

        def start_idx(c):
            return pltpu.async_copy(
                an_hbm.at[pl.ds(base + c * _CHUNK, _CHUNK)],
                bufs[c % _NBUF],
                sems_i[c % _NBUF],
            )

        def start_gather(c):
            return pltpu.async_copy(
                tabs.at[bufs[c % _NBUF].at[pl.ds(_SPLIT, nstream)]],
                vals[c % _NBUF],
                sems_g[c % _NBUF],
            )

        def process_hist(buf):
            def body(i, carry):
                start_i = i * (_UNROLL * _LANES)
                idxs = [
                    buf[pl.ds(start_i + u * _LANES, _LANES)]
                    for u in range(_UNROLL)
                ]
                addrs = [ix * _LANES + lanes for ix in idxs]
                for a in addrs:
                    plsc.addupdate_scatter(hist, [a], ones)
                return carry

            lax.fori_loop(0, _SPLIT // (_UNROLL * _LANES), body, 0)

        def accum(v, accs):
            def body(i, accs):
                loads = [
                    v[pl.ds((i * _NACC + u) * _LANES, _LANES)]
                    for u in range(_NACC)
                ]
                return tuple(a + x for a, x in zip(accs, loads))

            return lax.fori_loop(0, nstream // (_NACC * _LANES), body, accs)

        accs = tuple(jnp.zeros((_LANES,), jnp.float32) for _ in range(_NACC))

        hidx = [start_idx(0)]
        hg = {}
        for c in range(nchunks):
            if c + 1 < nchunks:
                hidx.append(start_idx(c + 1))
            hidx[c].wait()
            hg[c] = start_gather(c)
            process_hist(bufs[c % _NBUF])
            hg[c].wait()
            accs = accum(vals[c % _NBUF], accs)

        acc = accs[0]
        for a in accs[1:]:
            acc = acc + a
        for b in range(nrefs):
            acc = acc + hist[pl.ds(b * _LANES, _LANES)] * tab_v[pl.ds(b * _LANES, _LANES)]
        acc_v[...] = acc
        pltpu.sync_copy(acc_v, out_hbm.at[wid])

    return k(atomic_numbers, table_bcast, table_pad)


def _tc_combine(partials_ref, te_ref, out_ref):
    out_ref[...] = te_ref[...] - jnp.sum(partials_ref[...])


def kernel(total_energy, atomic_numbers, per_atom_references):
    an = atomic_numbers.astype(jnp.int32)
    nrefs = per_atom_references.shape[0]
    table_f32 = per_atom_references.astype(jnp.float32)
    table_bcast = jnp.broadcast_to(
        table_f32[:, None], (nrefs, _LANES)
    ).reshape(nrefs * _LANES)
    npad = -(-nrefs // _LANES) * _LANES
    table_pad = jnp.pad(table_f32, (0, npad - nrefs))

    partials = _sc_partial_sums(an, table_bcast, table_pad)

    return pl.pallas_call(
        _tc_combine,
        out_shape=jax.ShapeDtypeStruct(total_energy.shape, jnp.float32),
    )(partials, total_energy)


# speedup vs baseline: 1.0357x; 1.0009x over previous
import functools

import jax
import jax.numpy as jnp
from jax import lax
from jax.experimental import pallas as pl
from jax.experimental.pallas import tpu as pltpu
from jax.experimental.pallas import tpu_sc as plsc

_LANES = 16
_NWORKERS = 32
_CHUNK = 32768
_SPLIT = 29696
_NBUF = 2
_UNROLL = 16
_NACC = 8


def _sc_partial_sums(atomic_numbers, table_bcast, table_pad):
    natoms = atomic_numbers.shape[0]
    nrefs = table_bcast.shape[0] // _LANES
    npad = table_pad.shape[0]
    per_w = natoms // _NWORKERS
    nchunks = per_w // _CHUNK
    nstream = _CHUNK - _SPLIT
    assert per_w % _CHUNK == 0 and nchunks >= 2

    mesh = plsc.VectorSubcoreMesh(core_axis_name="c", subcore_axis_name="s")

    @functools.partial(
        pl.kernel,
        mesh=mesh,
        out_type=jax.ShapeDtypeStruct((_NWORKERS, _LANES), jnp.float32),
        compiler_params=pltpu.CompilerParams(
            use_tc_tiling_on_sc=False, needs_layout_passes=False
        ),
        scratch_types=[
            [pltpu.VMEM((_CHUNK,), jnp.int32) for _ in range(_NBUF)],
            [pltpu.VMEM((nstream,), jnp.float32) for _ in range(_NBUF)],
            pltpu.VMEM((nrefs * _LANES,), jnp.float32),
            pltpu.VMEM((nrefs * _LANES,), jnp.float32),
            pltpu.VMEM_SHARED((npad,), jnp.float32),
            pltpu.VMEM((_LANES,), jnp.float32),
            [pltpu.SemaphoreType.DMA for _ in range(_NBUF)],
            [pltpu.SemaphoreType.DMA for _ in range(_NBUF)],
        ],
    )
    def k(an_hbm, tabb_hbm, tabp_hbm, out_hbm, bufs, vals, hist, tab_v,
          tabs, acc_v, sems_i, sems_g):
        wid = lax.axis_index("s") * 2 + lax.axis_index("c")
        base = wid * per_w

        @pl.when(lax.axis_index("s") == 0)
        def _():
            pltpu.sync_copy(tabp_hbm, tabs)

        pltpu.sync_copy(tabb_hbm, tab_v)
        zeros = jnp.zeros((_LANES,), jnp.float32)
        for b in range(nrefs):
            hist[pl.ds(b * _LANES, _LANES)] = zeros
        plsc.subcore_barrier()

        lanes = lax.iota(jnp.int32, _LANES)
        ones = jnp.ones((_LANES,), jnp.float32)

        def start_idx(c):
            return pltpu.async_copy(
                an_hbm.at[pl.ds(base + c * _CHUNK, _CHUNK)],
                bufs[c % _NBUF],
                sems_i[c % _NBUF],
            )

        def start_gather(c):
            return pltpu.async_copy(
                tabs.at[bufs[c % _NBUF].at[pl.ds(_SPLIT, nstream)]],
                vals[c % _NBUF],
                sems_g[c % _NBUF],
            )

        def process_hist(buf):
            def body(i, carry):
                start_i = i * (_UNROLL * _LANES)
                idxs = [
                    buf[pl.ds(start_i + u * _LANES, _LANES)]
                    for u in range(_UNROLL)
                ]
                addrs = [ix * _LANES + lanes for ix in idxs]
                for a in addrs:
                    plsc.addupdate_scatter(hist, [a], ones)
                return carry

            lax.fori_loop(0, _SPLIT // (_UNROLL * _LANES), body, 0)

        def accum(v, accs):
            def body(i, accs):
                loads = [
                    v[pl.ds((i * _NACC + u) * _LANES, _LANES)]
                    for u in range(_NACC)
                ]
                return tuple(a + x for a, x in zip(accs, loads))

            return lax.fori_loop(0, nstream // (_NACC * _LANES), body, accs)

        accs = tuple(jnp.zeros((_LANES,), jnp.float32) for _ in range(_NACC))

        hidx = [start_idx(0)]
        hg = {}
        for c in range(nchunks):
            if c + 1 < nchunks:
                hidx.append(start_idx(c + 1))
            hidx[c].wait()
            hg[c] = start_gather(c)
            process_hist(bufs[c % _NBUF])
            hg[c].wait()
            accs = accum(vals[c % _NBUF], accs)

        acc = accs[0]
        for a in accs[1:]:
            acc = acc + a
        for b in range(nrefs):
            acc = acc + hist[pl.ds(b * _LANES, _LANES)] * tab_v[pl.ds(b * _LANES, _LANES)]
        acc_v[...] = acc
        pltpu.sync_copy(acc_v, out_hbm.at[wid])

    return k(atomic_numbers, table_bcast, table_pad)


def _tc_combine(partials_ref, te_ref, out_ref):
    out_ref[...] = te_ref[...] - jnp.sum(partials_ref[...])


def kernel(total_energy, atomic_numbers, per_atom_references):
    an = atomic_numbers.astype(jnp.int32)
    nrefs = per_atom_references.shape[0]
    table_f32 = per_atom_references.astype(jnp.float32)
    table_bcast = jnp.broadcast_to(
        table_f32[:, None], (nrefs, _LANES)
    ).reshape(nrefs * _LANES)
    npad = -(-nrefs // _LANES) * _LANES
    table_pad = jnp.pad(table_f32, (0, npad - nrefs))

    partials = _sc_partial_sums(an, table_bcast, table_pad)

    return pl.pallas_call(
        _tc_combine,
        out_shape=jax.ShapeDtypeStruct(total_energy.shape, jnp.float32),
    )(partials, total_energy)
